# baseline (speedup 1.0000x reference)
import functools

import jax
import jax.numpy as jnp
from jax import lax
from jax.experimental import pallas as pl
from jax.experimental.pallas import tpu as pltpu
from jax.experimental.pallas import tpu_sc as plsc

CUSTOMER_VOCAB = 1000001
AGE_VOCAB = 101
EMBED_DIM = 32
BATCH = 16384

_INFO = plsc.get_sparse_core_info()
_NC = _INFO.num_cores
_NS = _INFO.num_subcores
_NW = _NC * _NS
_BPW = BATCH // _NW
_CHUNK = 128
_NCHUNK = _BPW // _CHUNK

_mesh = plsc.VectorSubcoreMesh(core_axis_name="c", subcore_axis_name="s")


@functools.partial(
    pl.kernel,
    mesh=_mesh,
    out_type=jax.ShapeDtypeStruct((BATCH, 2 * EMBED_DIM), jnp.float32),
    scratch_types=[
        pltpu.VMEM((_NCHUNK, _CHUNK), jnp.int32),
        pltpu.VMEM((_NCHUNK, _CHUNK), jnp.int32),
        pltpu.VMEM((_BPW, EMBED_DIM), jnp.float32),
        pltpu.VMEM((_BPW, EMBED_DIM), jnp.float32),
        pltpu.SemaphoreType.DMA,
        pltpu.SemaphoreType.DMA,
    ],
    compiler_params=pltpu.CompilerParams(use_tc_tiling_on_sc=False),
)
def _gather_concat(cust_id_hbm, age_id_hbm, cust_tab_hbm, age_tab_hbm,
                   out_hbm, idx_c, idx_a, rows_c, rows_a, sem_c, sem_a):
    wid = lax.axis_index("s") * _NC + lax.axis_index("c")
    base = wid * _BPW

    for j in range(_NCHUNK):
        off = base + j * _CHUNK
        pltpu.sync_copy(cust_id_hbm.at[pl.ds(off, _CHUNK)], idx_c.at[j])
        pltpu.sync_copy(age_id_hbm.at[pl.ds(off, _CHUNK)], idx_a.at[j])

    copies = []
    for j in range(_NCHUNK):
        dst = rows_c.at[pl.ds(j * _CHUNK, _CHUNK)]
        copies.append(pltpu.async_copy(cust_tab_hbm.at[idx_c.at[j]], dst, sem_c))
    for j in range(_NCHUNK):
        dst = rows_a.at[pl.ds(j * _CHUNK, _CHUNK)]
        copies.append(pltpu.async_copy(age_tab_hbm.at[idx_a.at[j]], dst, sem_a))
    for c in copies:
        c.wait()

    pltpu.sync_copy(rows_c, out_hbm.at[pl.ds(base, _BPW), pl.ds(0, EMBED_DIM)])
    pltpu.sync_copy(rows_a,
                    out_hbm.at[pl.ds(base, _BPW), pl.ds(EMBED_DIM, EMBED_DIM)])


def kernel(customer_id, age, customer_table, age_table):
    return _gather_concat(customer_id, age, customer_table, age_table)


# speedup vs baseline: 1.0009x; 1.0009x over previous
import functools

import jax
import jax.numpy as jnp
from jax import lax
from jax.experimental import pallas as pl
from jax.experimental.pallas import tpu as pltpu
from jax.experimental.pallas import tpu_sc as plsc

CUSTOMER_VOCAB = 1000001
AGE_VOCAB = 101
EMBED_DIM = 32
BATCH = 16384

_INFO = plsc.get_sparse_core_info()
_NC = _INFO.num_cores
_NS = _INFO.num_subcores
_NW = _NC * _NS
_BPW = BATCH // _NW
_CHUNK = 128
_NCHUNK = _BPW // _CHUNK

_mesh = plsc.VectorSubcoreMesh(core_axis_name="c", subcore_axis_name="s")


@functools.partial(
    pl.kernel,
    mesh=_mesh,
    out_type=jax.ShapeDtypeStruct((BATCH, 2 * EMBED_DIM), jnp.float32),
    scratch_types=[
        pltpu.VMEM((_NCHUNK, _CHUNK), jnp.int32),
        pltpu.VMEM((_NCHUNK, _CHUNK), jnp.int32),
        pltpu.VMEM((_BPW, EMBED_DIM), jnp.float32),
        pltpu.VMEM((_BPW, EMBED_DIM), jnp.float32),
        pltpu.SemaphoreType.DMA,
        pltpu.SemaphoreType.DMA,
    ],
    compiler_params=pltpu.CompilerParams(use_tc_tiling_on_sc=False),
)
def _gather_concat(cust_id_hbm, age_id_hbm, cust_tab_hbm, age_tab_hbm,
                   out_hbm, idx_c, idx_a, rows_c, rows_a, sem_c, sem_a):
    wid = lax.axis_index("s") * _NC + lax.axis_index("c")
    base = wid * _BPW

    for j in range(_NCHUNK):
        off = base + j * _CHUNK
        pltpu.sync_copy(cust_id_hbm.at[pl.ds(off, _CHUNK)], idx_c.at[j])
        pltpu.sync_copy(age_id_hbm.at[pl.ds(off, _CHUNK)], idx_a.at[j])

    copies = []
    for j in range(_NCHUNK):
        dst = rows_c.at[pl.ds(j * _CHUNK, _CHUNK)]
        copies.append(pltpu.async_copy(cust_tab_hbm.at[idx_c.at[j]], dst, sem_c))
    for j in range(_NCHUNK):
        dst = rows_a.at[pl.ds(j * _CHUNK, _CHUNK)]
        copies.append(pltpu.async_copy(age_tab_hbm.at[idx_a.at[j]], dst, sem_a))
    for c in copies:
        c.wait()

    pltpu.sync_copy(rows_c, out_hbm.at[pl.ds(base, _BPW), pl.ds(0, EMBED_DIM)])
    pltpu.sync_copy(rows_a,
                    out_hbm.at[pl.ds(base, _BPW), pl.ds(EMBED_DIM, EMBED_DIM)])


def kernel(customer_id, age, customer_table, age_table):
    customer_table = customer_table.reshape(-1).reshape(CUSTOMER_VOCAB,
                                                        EMBED_DIM)
    age_table = age_table.reshape(-1).reshape(AGE_VOCAB, EMBED_DIM)
    return _gather_concat(customer_id, age, customer_table, age_table)
